# baseline (speedup 1.0000x reference)
import functools

import jax
import jax.numpy as jnp
from jax import lax
from jax.experimental import pallas as pl
from jax.experimental.pallas import tpu as pltpu
from jax.experimental.pallas import tpu_sc as plsc

VOCAB = 100000
DIM = 256
BATCH = 1024
SEQ = 50

NC = 2
NS = 16
NW = NC * NS
BPW = BATCH // NW
LANES = 16

_MESH = plsc.VectorSubcoreMesh(core_axis_name="c", subcore_axis_name="s")


HALF = 128
SEQ2 = 2 * SEQ


@functools.partial(
    pl.kernel,
    mesh=_MESH,
    out_type=jax.ShapeDtypeStruct((BATCH, DIM), jnp.float32),
    scratch_types=[
        pltpu.VMEM((BPW, SEQ2), jnp.int32),
        pltpu.VMEM((SEQ2, HALF), jnp.float32),
        pltpu.VMEM((1, DIM), jnp.float32),
        pltpu.SemaphoreType.DMA,
    ],
)
def _pool(idx_hbm, table_hbm, h_hbm, idx_v, rows_v, hrow_v, sem):
    wid = lax.axis_index("s") * NC + lax.axis_index("c")
    base = wid * BPW
    pltpu.sync_copy(idx_hbm.at[pl.ds(base, BPW)], idx_v)

    def row_body(r, carry):
        pltpu.async_copy(table_hbm.at[idx_v.at[r]], rows_v, sem).wait()
        for c in range(HALF // LANES):
            def lo(j, acc):
                return acc + rows_v[2 * j, pl.ds(c * LANES, LANES)]
            def hi(j, acc):
                return acc + rows_v[2 * j + 1, pl.ds(c * LANES, LANES)]
            acc_lo = lax.fori_loop(0, SEQ, lo, jnp.zeros((LANES,), jnp.float32))
            acc_hi = lax.fori_loop(0, SEQ, hi, jnp.zeros((LANES,), jnp.float32))
            hrow_v[0, pl.ds(c * LANES, LANES)] = acc_lo * (1.0 / SEQ)
            hrow_v[0, pl.ds(HALF + c * LANES, LANES)] = acc_hi * (1.0 / SEQ)
        pltpu.sync_copy(hrow_v, h_hbm.at[pl.ds(base + r, 1)])
        return carry

    lax.fori_loop(0, BPW, row_body, 0)


def _mm_body(h_ref, w_ref, b_ref, out_ref):
    out_ref[...] = (
        lax.dot_general(
            h_ref[...], w_ref[...],
            (((1,), (1,)), ((), ())),
            preferred_element_type=jnp.float32,
        )
        + b_ref[...]
    )


def _project(h, proj_weight, proj_bias, bn=2048):
    nblk = (VOCAB + bn - 1) // bn
    return pl.pallas_call(
        _mm_body,
        grid=(nblk,),
        in_specs=[
            pl.BlockSpec((BATCH, DIM), lambda i: (0, 0)),
            pl.BlockSpec((bn, DIM), lambda i: (i, 0)),
            pl.BlockSpec((1, bn), lambda i: (0, i)),
        ],
        out_specs=pl.BlockSpec((BATCH, bn), lambda i: (0, i)),
        out_shape=jax.ShapeDtypeStruct((BATCH, VOCAB), jnp.float32),
    )(h, proj_weight, proj_bias.reshape(1, VOCAB))


@jax.jit
def kernel(indices, embed_weight, proj_weight, proj_bias):
    idx = indices.astype(jnp.int32)
    idx2 = jnp.stack([2 * idx, 2 * idx + 1], axis=-1).reshape(BATCH, SEQ2)
    table2 = embed_weight.reshape(2 * VOCAB, HALF)
    h = _pool(idx2, table2)
    return _project(h, proj_weight, proj_bias)
